# baseline (speedup 1.0000x reference)
import functools

import jax
import jax.numpy as jnp
from jax import lax
from jax.experimental import pallas as pl
from jax.experimental.pallas import tpu as pltpu
from jax.experimental.pallas import tpu_sc as plsc

B = 4096
N = 20
S = 20
E = 32
R = B * S

NC, NS = 2, 16
NW = NC * NS
BPW = B // NW
CB = 8
NCHUNK = BPW // CB


def _sc_body(idx_hbm, table_hbm, out_hbm, idx_v, jq_v, r0_v, r1_v, out_v,
             sem0, sem1):
    wid = lax.axis_index("s") * NC + lax.axis_index("c")
    wb = wid * BPW
    bufs = (r0_v, r1_v)
    sems = (sem0, sem1)

    def xform(k, carry):
        for bb in range(CB):
            w = idx_v[bb, pl.ds(k * 16, 16)]
            jq_v[bb, pl.ds(k * 16, 16)] = w >> 2
        return carry

    def fire(bb, buf):
        return [
            pltpu.async_copy(
                table_hbm.at[jq_v.at[bb, pl.ds(g * 128, 128)]],
                bufs[buf].at[pl.ds(g * 128, 128), :],
                sems[buf],
            )
            for g in range(3)
        ]

    def chunk_body(c, carry):
        b0 = wb + c * CB
        pltpu.sync_copy(idx_hbm.at[pl.ds(b0, CB), :], idx_v)
        lax.fori_loop(0, 25, xform, 0)
        descs = {0: fire(0, 0)}
        for bb in range(CB):
            if bb < CB - 1:
                descs[bb + 1] = fire(bb + 1, (bb + 1) % 2)
            for d in descs[bb]:
                d.wait()
            out_v[bb * S, pl.ds(0, 16)] = bufs[bb % 2][0, pl.ds(0, 16)]
            out_v[bb * S, pl.ds(16, 16)] = bufs[bb % 2][1, pl.ds(0, 16)]
        pltpu.sync_copy(out_v, out_hbm.at[pl.ds(b0 * S, CB * S), :])
        return carry

    lax.fori_loop(0, NCHUNK, chunk_body, 0)


@functools.cache
def _sc_call():
    return functools.partial(
        pl.kernel,
        out_type=jax.ShapeDtypeStruct((R, E), jnp.float32),
        mesh=plsc.VectorSubcoreMesh(
            core_axis_name="c", subcore_axis_name="s",
            num_cores=NC, num_subcores=NS,
        ),
        scratch_types=[
            pltpu.VMEM((CB, N * S), jnp.int32),
            pltpu.VMEM((CB, N * S), jnp.int32),
            pltpu.VMEM((384, 128), jnp.float32),
            pltpu.VMEM((384, 128), jnp.float32),
            pltpu.VMEM((CB * S, E), jnp.float32),
            pltpu.SemaphoreType.DMA,
            pltpu.SemaphoreType.DMA,
        ],
        compiler_params=pltpu.CompilerParams(use_tc_tiling_on_sc=True),
    )(_sc_body)


def kernel(x, table):
    t128 = table.reshape(250000, 128)
    out = _sc_call()(x.reshape(B, N * S), t128)
    return out.reshape(B, 1, S, E)


# speedup vs baseline: 1.2540x; 1.2540x over previous
import functools

import jax
import jax.numpy as jnp
from jax import lax
from jax.experimental import pallas as pl
from jax.experimental.pallas import tpu as pltpu
from jax.experimental.pallas import tpu_sc as plsc

B = 4096
N = 20
S = 20
E = 32

NC, NS = 2, 16
NW = NC * NS
BPW = B // NW
CB = 4
CI = CB * N * S
CR = CB * S
GSZ = 80
GPC = CI // GSZ
NCHUNK = BPW // CB


def _sc_body(idx_hbm, table_hbm, out_hbm, idx_v, rows_v, out_v, sem):
    wid = lax.axis_index("s") * NC + lax.axis_index("c")
    wb = wid * BPW

    def acc_body(r, carry):
        bb = r // S
        s = r - bb * S
        base = bb * (N * S) + s
        a0 = rows_v[base, pl.ds(0, 16)]
        a1 = rows_v[base, pl.ds(16, 16)]
        for n in range(1, N):
            a0 = a0 + rows_v[base + n * S, pl.ds(0, 16)]
            a1 = a1 + rows_v[base + n * S, pl.ds(16, 16)]
        out_v[bb, s, pl.ds(0, 16)] = a0
        out_v[bb, s, pl.ds(16, 16)] = a1
        return carry

    def chunk_body(c, carry):
        b0 = wb + c * CB
        pltpu.sync_copy(idx_hbm.at[pl.ds(b0, CB), :], idx_v)
        descs = [
            pltpu.async_copy(
                table_hbm.at[idx_v.at[bb, pl.ds(g * GSZ, GSZ)]],
                rows_v.at[pl.ds(bb * (N * S) + g * GSZ, GSZ), :],
                sem,
            )
            for bb in range(CB)
            for g in range(N * S // GSZ)
        ]
        for d in descs:
            d.wait()
        lax.fori_loop(0, CR, acc_body, 0)
        pltpu.sync_copy(out_v, out_hbm.at[pl.ds(b0, CB), 0, :, :])
        return carry

    lax.fori_loop(0, NCHUNK, chunk_body, 0)


@functools.cache
def _sc_call():
    return functools.partial(
        pl.kernel,
        out_type=jax.ShapeDtypeStruct((B, 1, S, E), jnp.float32),
        mesh=plsc.VectorSubcoreMesh(
            core_axis_name="c", subcore_axis_name="s",
            num_cores=NC, num_subcores=NS,
        ),
        scratch_types=[
            pltpu.VMEM((CB, N * S), jnp.int32),
            pltpu.VMEM((CI, E), jnp.float32),
            pltpu.VMEM((CB, S, E), jnp.float32),
            pltpu.SemaphoreType.DMA,
        ],
        compiler_params=pltpu.CompilerParams(use_tc_tiling_on_sc=False),
    )(_sc_body)


def kernel(x, table):
    return _sc_call()(x.reshape(B, N * S), table)
